# baseline (speedup 1.0000x reference)
import functools

import jax
import jax.numpy as jnp
from jax import lax
from jax.experimental import pallas as pl
from jax.experimental.pallas import tpu as pltpu
from jax.experimental.pallas import tpu_sc as plsc

N_NODES = 10000
N_EDGES = 320000
D_IN = 128
NC, NS = 2, 16
NW = NC * NS
CH = 80
NCHUNKS = N_EDGES // CH
KSUB = 25
NSTEPS = NCHUNKS // KSUB
N_ACC = 10240
RPT = N_ACC // NS
N_DEN = 10000
LANES = 16
DSHIFT = 14


def _sc_gather_scatter(x, idx3d, zeros_pad):
    mesh = plsc.VectorSubcoreMesh(core_axis_name="c", subcore_axis_name="s")

    @functools.partial(
        pl.kernel,
        out_type=(
            jax.ShapeDtypeStruct((NC, N_ACC, D_IN), jnp.float32),
            jax.ShapeDtypeStruct((NW, N_DEN), jnp.float32),
        ),
        mesh=mesh,
        scratch_types=[
            pltpu.VMEM((CH, D_IN), jnp.float32),
            pltpu.VMEM((CH, D_IN), jnp.float32),
            pltpu.VMEM((CH,), jnp.int32),
            pltpu.VMEM((CH,), jnp.int32),
            pltpu.VMEM((CH,), jnp.int32),
            pltpu.VMEM((CH,), jnp.int32),
            pltpu.VMEM((N_DEN,), jnp.float32),
            pltpu.VMEM_SHARED((N_ACC, D_IN), jnp.float32),
            pltpu.SemaphoreType.DMA,
            pltpu.SemaphoreType.DMA,
            pltpu.SemaphoreType.DMA,
            pltpu.SemaphoreType.DMA,
        ],
        compiler_params=pltpu.CompilerParams(needs_layout_passes=False),
    )
    def k(x_hbm, idx_hbm, zero_hbm, num_hbm, den_hbm,
          rows0, rows1, si0, si1, di0, di1, den_v, acc_sh,
          gs0, gs1, ss0, ss1):
        rows = (rows0, rows1)
        sib = (si0, si1)
        dib = (di0, di1)
        gsem = (gs0, gs1)
        ssem = (ss0, ss1)
        cid = lax.axis_index("c")
        sid = lax.axis_index("s")
        wid = cid * NS + sid

        pltpu.sync_copy(zero_hbm.at[pl.ds(sid * RPT, RPT)],
                        acc_sh.at[pl.ds(sid * RPT, RPT)])
        z16 = jnp.zeros((LANES,), jnp.float32)

        @pl.loop(0, N_DEN, step=LANES)
        def _(i):
            den_v[pl.ds(i, LANES)] = z16

        plsc.subcore_barrier()

        zi16 = jnp.zeros((LANES,), jnp.int32)

        def body(iv):
            def unpack(b):
                for g in range(CH // LANES):
                    gsl = pl.ds(g * LANES, LANES)
                    p = iv[0, b, gsl]
                    sib[b % 2][gsl] = p & ((1 << DSHIFT) - 1)
                    dib[b % 2][gsl] = lax.shift_right_logical(p, DSHIFT)

            def den_upd(b):
                for g in range(CH // LANES):
                    gsl = pl.ds(g * LANES, LANES)
                    plsc.addupdate_scatter(
                        den_v, [dib[b % 2][gsl]],
                        plsc.bitcast(iv[0, KSUB + b, gsl], jnp.float32))

            def scale(b, r):
                @pl.loop(0, CH, step=2)
                def _(c):
                    a0 = plsc.bitcast(plsc.load_gather(
                        iv, [zi16, jnp.full((LANES,), KSUB + b, jnp.int32),
                             jnp.full((LANES,), c, jnp.int32)]), jnp.float32)
                    a1 = plsc.bitcast(plsc.load_gather(
                        iv, [zi16, jnp.full((LANES,), KSUB + b, jnp.int32),
                             jnp.full((LANES,), c + 1, jnp.int32)]), jnp.float32)
                    for j in range(D_IN // LANES):
                        sl = pl.ds(j * LANES, LANES)
                        r[c, sl] = r[c, sl] * a0
                        r[c + 1, sl] = r[c + 1, sl] * a1

            def gath(b):
                return pltpu.async_copy(
                    x_hbm.at[sib[b % 2]], rows[b % 2], gsem[b % 2])

            def scat(b):
                return pltpu.async_copy(
                    rows[b % 2], acc_sh.at[dib[b % 2]], ssem[b % 2],
                    add=True)

            gd = [None] * KSUB
            sd = [None] * KSUB
            for b in range(2):
                unpack(b)
                gd[b] = gath(b)
            for b in range(2):
                den_upd(b)

            gd[0].wait(); scale(0, rows[0]); sd[0] = scat(0)
            for b in range(1, KSUB):
                gd[b].wait(); scale(b, rows[b % 2]); sd[b] = scat(b)
                nb = b + 1
                if 2 <= nb < KSUB:
                    sd[nb - 2].wait()
                    unpack(nb)
                    gd[nb] = gath(nb)
                    den_upd(nb)
            for b in range(KSUB - 2, KSUB):
                sd[b].wait()

        pltpu.emit_pipeline(
            body,
            grid=(NSTEPS,),
            in_specs=[
                pl.BlockSpec((1, 2 * KSUB, CH), lambda i: (i, 0, 0)),
            ],
            out_specs=[],
            core_axis_name=("c", "s"),
            dimension_semantics=(pltpu.PARALLEL,),
        )(idx_hbm)

        plsc.subcore_barrier()
        pltpu.sync_copy(acc_sh.at[pl.ds(sid * RPT, RPT)],
                        num_hbm.at[cid, pl.ds(sid * RPT, RPT)])
        pltpu.sync_copy(den_v, den_hbm.at[wid])

    return k(x, idx3d, zeros_pad)


def _attn_mlp2(ea_t, W1t_a, b1c_a, W2c_a, b2_a, W1t_b, b1c_b, W2c_b, b2_b):
    BE = 32000

    def body(ea_ref, W1a, b1a, W2a, b2a, W1b, b1b, W2b, b2b, oa_ref, ob_ref):
        ea = ea_ref[...]
        for W1, b1, W2, b2, o_ref in ((W1a, b1a, W2a, b2a, oa_ref),
                                      (W1b, b1b, W2b, b2b, ob_ref)):
            h = jnp.maximum(
                jnp.dot(W1[...], ea,
                        preferred_element_type=jnp.float32) + b1[...], 0.0)
            z = jnp.sum(h * W2[...], axis=0, keepdims=True) + b2[...]
            o_ref[...] = 1.0 / (1.0 + jnp.exp(-z))

    wspecs = [
        pl.BlockSpec((32, 16), lambda i: (0, 0)),
        pl.BlockSpec((32, 1), lambda i: (0, 0)),
        pl.BlockSpec((32, 1), lambda i: (0, 0)),
        pl.BlockSpec((1, 1), lambda i: (0, 0)),
    ]
    return pl.pallas_call(
        body,
        grid=(N_EDGES // BE,),
        in_specs=[pl.BlockSpec((16, BE), lambda i: (0, i))] + wspecs + wspecs,
        out_specs=[pl.BlockSpec((1, BE), lambda i: (0, i))] * 2,
        out_shape=[jax.ShapeDtypeStruct((1, N_EDGES), jnp.float32)] * 2,
    )(ea_t, W1t_a, b1c_a, W2c_a, b2_a, W1t_b, b1c_b, W2c_b, b2_b)


BN = 1024


def _layer_mid(num_p, den_p, x, sW, sb, nW, nb, sW2, sb2):
    def body(a_ref, b_ref, dp_ref, x_ref, sW_ref, sb_ref, nW_ref, nb_ref,
             sW2_ref, sb2_ref, h_ref, hs_ref):
        den = jnp.sum(dp_ref[...], axis=0)[:, None] + 1e-8
        agg = (a_ref[0] + b_ref[0]) / den
        xs = jnp.dot(x_ref[...], sW_ref[...],
                     preferred_element_type=jnp.float32) + sb_ref[...]
        h = jnp.maximum(
            xs + jnp.dot(agg, nW_ref[...],
                         preferred_element_type=jnp.float32)
            + nb_ref[...], 0.0)
        h_ref[...] = h
        hs_ref[...] = jnp.dot(h, sW2_ref[...],
                              preferred_element_type=jnp.float32) + sb2_ref[...]

    return pl.pallas_call(
        body,
        grid=(pl.cdiv(N_NODES, BN),),
        in_specs=[
            pl.BlockSpec((1, BN, D_IN), lambda i: (0, i, 0)),
            pl.BlockSpec((1, BN, D_IN), lambda i: (1, i, 0)),
            pl.BlockSpec((NW, BN), lambda i: (0, i)),
            pl.BlockSpec((BN, D_IN), lambda i: (i, 0)),
            pl.BlockSpec((D_IN, D_IN), lambda i: (0, 0)),
            pl.BlockSpec((1, D_IN), lambda i: (0, 0)),
            pl.BlockSpec((D_IN, D_IN), lambda i: (0, 0)),
            pl.BlockSpec((1, D_IN), lambda i: (0, 0)),
            pl.BlockSpec((D_IN, D_IN), lambda i: (0, 0)),
            pl.BlockSpec((1, D_IN), lambda i: (0, 0)),
        ],
        out_specs=[
            pl.BlockSpec((BN, D_IN), lambda i: (i, 0)),
            pl.BlockSpec((BN, D_IN), lambda i: (i, 0)),
        ],
        out_shape=[
            jax.ShapeDtypeStruct((N_NODES, D_IN), jnp.float32),
            jax.ShapeDtypeStruct((N_NODES, D_IN), jnp.float32),
        ],
    )(num_p, num_p, den_p, x, sW, sb, nW, nb, sW2, sb2)


def _layer_post(num_p, den_p, hs, nW, nb, cW, cb):
    def body(a_ref, b_ref, dp_ref, hs_ref, nW_ref, nb_ref, cW_ref, cb_ref,
             o_ref):
        den = jnp.sum(dp_ref[...], axis=0)[:, None] + 1e-8
        agg = (a_ref[0] + b_ref[0]) / den
        h = jnp.maximum(
            hs_ref[...] + jnp.dot(agg, nW_ref[...],
                                  preferred_element_type=jnp.float32)
            + nb_ref[...], 0.0)
        o_ref[...] = jnp.dot(h, cW_ref[...],
                             preferred_element_type=jnp.float32) + cb_ref[...]

    return pl.pallas_call(
        body,
        grid=(pl.cdiv(N_NODES, BN),),
        in_specs=[
            pl.BlockSpec((1, BN, D_IN), lambda i: (0, i, 0)),
            pl.BlockSpec((1, BN, D_IN), lambda i: (1, i, 0)),
            pl.BlockSpec((NW, BN), lambda i: (0, i)),
            pl.BlockSpec((BN, D_IN), lambda i: (i, 0)),
            pl.BlockSpec((D_IN, D_IN), lambda i: (0, 0)),
            pl.BlockSpec((1, D_IN), lambda i: (0, 0)),
            pl.BlockSpec((D_IN, 1), lambda i: (0, 0)),
            pl.BlockSpec((1, 1), lambda i: (0, 0)),
        ],
        out_specs=pl.BlockSpec((BN, 1), lambda i: (i, 0)),
        out_shape=jax.ShapeDtypeStruct((N_NODES, 1), jnp.float32),
    )(num_p, num_p, den_p, hs, nW, nb, cW, cb)


def kernel(x, edge_index, edge_attr,
           e1_W1, e1_b1, e1_W2, e1_b2, s1_W, s1_b, n1_W, n1_b,
           e2_W1, e2_b1, e2_W2, e2_b2, s2_W, s2_b, n2_W, n2_b,
           cls_W, cls_b):
    src3d = edge_index[0].reshape(NSTEPS, KSUB, CH)
    dst3d = edge_index[1].reshape(NSTEPS, KSUB, CH)
    zeros_pad = jnp.zeros((N_ACC, D_IN), jnp.float32)

    ea_t = edge_attr.T
    attn1, attn2 = _attn_mlp2(ea_t,
                              e1_W1.T, e1_b1.reshape(-1, 1), e1_W2,
                              e1_b2.reshape(1, 1),
                              e2_W1.T, e2_b1.reshape(-1, 1), e2_W2,
                              e2_b2.reshape(1, 1))
    packed = src3d | (dst3d << DSHIFT)
    a1bits = lax.bitcast_convert_type(
        attn1, jnp.int32).reshape(NSTEPS, KSUB, CH)
    a2bits = lax.bitcast_convert_type(
        attn2, jnp.int32).reshape(NSTEPS, KSUB, CH)
    idx1 = jnp.concatenate([packed, a1bits], axis=1)
    idx2 = jnp.concatenate([packed, a2bits], axis=1)

    num1, den1 = _sc_gather_scatter(x, idx1, zeros_pad)
    h, hs2 = _layer_mid(num1, den1, x, s1_W, s1_b.reshape(1, -1),
                        n1_W, n1_b.reshape(1, -1),
                        s2_W, s2_b.reshape(1, -1))

    num2, den2 = _sc_gather_scatter(h, idx2, zeros_pad)
    logits = _layer_post(num2, den2, hs2, n2_W, n2_b.reshape(1, -1),
                         cls_W, cls_b.reshape(1, 1))
    return logits[:, 0]


# speedup vs baseline: 1.2444x; 1.2444x over previous
import functools

import jax
import jax.numpy as jnp
from jax import lax
from jax.experimental import pallas as pl
from jax.experimental.pallas import tpu as pltpu
from jax.experimental.pallas import tpu_sc as plsc

N_NODES = 10000
N_EDGES = 320000
D_IN = 128
NC, NS = 2, 16
NW = NC * NS
CH = 80
NCHUNKS = N_EDGES // CH
KSUB = 5
NSTEPS = NCHUNKS // KSUB
N_ACC = 10240
RPT = N_ACC // NS
N_DEN = 10000
LANES = 16
DSHIFT = 14


def _sc_gather_scatter(x, idx3d, zeros_pad):
    mesh = plsc.VectorSubcoreMesh(core_axis_name="c", subcore_axis_name="s")

    @functools.partial(
        pl.kernel,
        out_type=(
            jax.ShapeDtypeStruct((NC, N_ACC, D_IN), jnp.float32),
            jax.ShapeDtypeStruct((NW, N_DEN), jnp.float32),
        ),
        mesh=mesh,
        scratch_types=[
            pltpu.VMEM((CH, D_IN), jnp.float32),
            pltpu.VMEM((CH, D_IN), jnp.float32),
            pltpu.VMEM((CH, D_IN), jnp.float32),
            pltpu.VMEM((CH,), jnp.int32),
            pltpu.VMEM((CH,), jnp.int32),
            pltpu.VMEM((CH,), jnp.int32),
            pltpu.VMEM((CH,), jnp.int32),
            pltpu.VMEM((CH,), jnp.int32),
            pltpu.VMEM((CH,), jnp.int32),
            pltpu.VMEM((N_DEN,), jnp.float32),
            pltpu.VMEM_SHARED((N_ACC, D_IN), jnp.float32),
            pltpu.SemaphoreType.DMA,
            pltpu.SemaphoreType.DMA,
            pltpu.SemaphoreType.DMA,
            pltpu.SemaphoreType.DMA,
            pltpu.SemaphoreType.DMA,
            pltpu.SemaphoreType.DMA,
        ],
        compiler_params=pltpu.CompilerParams(needs_layout_passes=False),
    )
    def k(x_hbm, idx_hbm, zero_hbm, num_hbm, den_hbm,
          rows0, rows1, rows2, si0, si1, si2,
          di0, di1, di2, den_v, acc_sh,
          gs0, gs1, gs2, ss0, ss1, ss2):
        rows = (rows0, rows1, rows2)
        sib = (si0, si1, si2)
        dib = (di0, di1, di2)
        gsem = (gs0, gs1, gs2)
        ssem = (ss0, ss1, ss2)
        cid = lax.axis_index("c")
        sid = lax.axis_index("s")
        wid = cid * NS + sid

        pltpu.sync_copy(zero_hbm.at[pl.ds(sid * RPT, RPT)],
                        acc_sh.at[pl.ds(sid * RPT, RPT)])
        z16 = jnp.zeros((LANES,), jnp.float32)

        @pl.loop(0, N_DEN, step=LANES)
        def _(i):
            den_v[pl.ds(i, LANES)] = z16

        plsc.subcore_barrier()

        zi16 = jnp.zeros((LANES,), jnp.int32)

        def body(iv):
            def unpack(b):
                for g in range(CH // LANES):
                    gsl = pl.ds(g * LANES, LANES)
                    p = iv[0, b, gsl]
                    sib[b % 3][gsl] = p & ((1 << DSHIFT) - 1)
                    dib[b % 3][gsl] = lax.shift_right_logical(p, DSHIFT)

            def den_upd(b):
                for g in range(CH // LANES):
                    gsl = pl.ds(g * LANES, LANES)
                    plsc.addupdate_scatter(
                        den_v, [dib[b % 3][gsl]],
                        plsc.bitcast(iv[0, KSUB + b, gsl], jnp.float32))

            def scale(b, r):
                @pl.loop(0, CH, step=2)
                def _(c):
                    a0 = plsc.bitcast(plsc.load_gather(
                        iv, [zi16, jnp.full((LANES,), KSUB + b, jnp.int32),
                             jnp.full((LANES,), c, jnp.int32)]), jnp.float32)
                    a1 = plsc.bitcast(plsc.load_gather(
                        iv, [zi16, jnp.full((LANES,), KSUB + b, jnp.int32),
                             jnp.full((LANES,), c + 1, jnp.int32)]), jnp.float32)
                    for j in range(D_IN // LANES):
                        sl = pl.ds(j * LANES, LANES)
                        r[c, sl] = r[c, sl] * a0
                        r[c + 1, sl] = r[c + 1, sl] * a1

            def gath(b):
                return pltpu.async_copy(
                    x_hbm.at[sib[b % 3]], rows[b % 3], gsem[b % 3])

            def scat(b):
                return pltpu.async_copy(
                    rows[b % 3], acc_sh.at[dib[b % 3]], ssem[b % 3],
                    add=True)

            gd = [None] * KSUB
            sd = [None] * KSUB
            for b in range(3):
                unpack(b)
                gd[b] = gath(b)
            for b in range(3):
                den_upd(b)

            gd[0].wait(); scale(0, rows[0]); sd[0] = scat(0)
            for b in range(1, KSUB):
                gd[b].wait(); scale(b, rows[b % 3]); sd[b] = scat(b)
                nb = b + 2
                if 3 <= nb < KSUB:
                    sd[nb - 3].wait()
                    unpack(nb)
                    gd[nb] = gath(nb)
                    den_upd(nb)
            for b in range(KSUB - 3, KSUB):
                sd[b].wait()

        pltpu.emit_pipeline(
            body,
            grid=(NSTEPS,),
            in_specs=[
                pl.BlockSpec((1, 2 * KSUB, CH), lambda i: (i, 0, 0)),
            ],
            out_specs=[],
            core_axis_name=("c", "s"),
            dimension_semantics=(pltpu.PARALLEL,),
        )(idx_hbm)

        plsc.subcore_barrier()
        pltpu.sync_copy(acc_sh.at[pl.ds(sid * RPT, RPT)],
                        num_hbm.at[cid, pl.ds(sid * RPT, RPT)])
        pltpu.sync_copy(den_v, den_hbm.at[wid])

    return k(x, idx3d, zeros_pad)


def _attn_mlp2(ea_t, W1t_a, b1c_a, W2c_a, b2_a, W1t_b, b1c_b, W2c_b, b2_b):
    BE = 32000

    def body(ea_ref, W1a, b1a, W2a, b2a, W1b, b1b, W2b, b2b, oa_ref, ob_ref):
        ea = ea_ref[...]
        for W1, b1, W2, b2, o_ref in ((W1a, b1a, W2a, b2a, oa_ref),
                                      (W1b, b1b, W2b, b2b, ob_ref)):
            h = jnp.maximum(
                jnp.dot(W1[...], ea,
                        preferred_element_type=jnp.float32) + b1[...], 0.0)
            z = jnp.sum(h * W2[...], axis=0, keepdims=True) + b2[...]
            o_ref[...] = 1.0 / (1.0 + jnp.exp(-z))

    wspecs = [
        pl.BlockSpec((32, 16), lambda i: (0, 0)),
        pl.BlockSpec((32, 1), lambda i: (0, 0)),
        pl.BlockSpec((32, 1), lambda i: (0, 0)),
        pl.BlockSpec((1, 1), lambda i: (0, 0)),
    ]
    return pl.pallas_call(
        body,
        grid=(N_EDGES // BE,),
        in_specs=[pl.BlockSpec((16, BE), lambda i: (0, i))] + wspecs + wspecs,
        out_specs=[pl.BlockSpec((1, BE), lambda i: (0, i))] * 2,
        out_shape=[jax.ShapeDtypeStruct((1, N_EDGES), jnp.float32)] * 2,
    )(ea_t, W1t_a, b1c_a, W2c_a, b2_a, W1t_b, b1c_b, W2c_b, b2_b)


BN = 1024


def _layer_mid(num_p, den_p, x, sW, sb, nW, nb, sW2, sb2):
    def body(a_ref, b_ref, dp_ref, x_ref, sW_ref, sb_ref, nW_ref, nb_ref,
             sW2_ref, sb2_ref, h_ref, hs_ref):
        den = jnp.sum(dp_ref[...], axis=0)[:, None] + 1e-8
        agg = (a_ref[0] + b_ref[0]) / den
        xs = jnp.dot(x_ref[...], sW_ref[...],
                     preferred_element_type=jnp.float32) + sb_ref[...]
        h = jnp.maximum(
            xs + jnp.dot(agg, nW_ref[...],
                         preferred_element_type=jnp.float32)
            + nb_ref[...], 0.0)
        h_ref[...] = h
        hs_ref[...] = jnp.dot(h, sW2_ref[...],
                              preferred_element_type=jnp.float32) + sb2_ref[...]

    return pl.pallas_call(
        body,
        grid=(pl.cdiv(N_NODES, BN),),
        in_specs=[
            pl.BlockSpec((1, BN, D_IN), lambda i: (0, i, 0)),
            pl.BlockSpec((1, BN, D_IN), lambda i: (1, i, 0)),
            pl.BlockSpec((NW, BN), lambda i: (0, i)),
            pl.BlockSpec((BN, D_IN), lambda i: (i, 0)),
            pl.BlockSpec((D_IN, D_IN), lambda i: (0, 0)),
            pl.BlockSpec((1, D_IN), lambda i: (0, 0)),
            pl.BlockSpec((D_IN, D_IN), lambda i: (0, 0)),
            pl.BlockSpec((1, D_IN), lambda i: (0, 0)),
            pl.BlockSpec((D_IN, D_IN), lambda i: (0, 0)),
            pl.BlockSpec((1, D_IN), lambda i: (0, 0)),
        ],
        out_specs=[
            pl.BlockSpec((BN, D_IN), lambda i: (i, 0)),
            pl.BlockSpec((BN, D_IN), lambda i: (i, 0)),
        ],
        out_shape=[
            jax.ShapeDtypeStruct((N_NODES, D_IN), jnp.float32),
            jax.ShapeDtypeStruct((N_NODES, D_IN), jnp.float32),
        ],
    )(num_p, num_p, den_p, x, sW, sb, nW, nb, sW2, sb2)


def _layer_post(num_p, den_p, hs, nW, nb, cW, cb):
    def body(a_ref, b_ref, dp_ref, hs_ref, nW_ref, nb_ref, cW_ref, cb_ref,
             o_ref):
        den = jnp.sum(dp_ref[...], axis=0)[:, None] + 1e-8
        agg = (a_ref[0] + b_ref[0]) / den
        h = jnp.maximum(
            hs_ref[...] + jnp.dot(agg, nW_ref[...],
                                  preferred_element_type=jnp.float32)
            + nb_ref[...], 0.0)
        o_ref[...] = jnp.dot(h, cW_ref[...],
                             preferred_element_type=jnp.float32) + cb_ref[...]

    return pl.pallas_call(
        body,
        grid=(pl.cdiv(N_NODES, BN),),
        in_specs=[
            pl.BlockSpec((1, BN, D_IN), lambda i: (0, i, 0)),
            pl.BlockSpec((1, BN, D_IN), lambda i: (1, i, 0)),
            pl.BlockSpec((NW, BN), lambda i: (0, i)),
            pl.BlockSpec((BN, D_IN), lambda i: (i, 0)),
            pl.BlockSpec((D_IN, D_IN), lambda i: (0, 0)),
            pl.BlockSpec((1, D_IN), lambda i: (0, 0)),
            pl.BlockSpec((D_IN, 1), lambda i: (0, 0)),
            pl.BlockSpec((1, 1), lambda i: (0, 0)),
        ],
        out_specs=pl.BlockSpec((BN, 1), lambda i: (i, 0)),
        out_shape=jax.ShapeDtypeStruct((N_NODES, 1), jnp.float32),
    )(num_p, num_p, den_p, hs, nW, nb, cW, cb)


def kernel(x, edge_index, edge_attr,
           e1_W1, e1_b1, e1_W2, e1_b2, s1_W, s1_b, n1_W, n1_b,
           e2_W1, e2_b1, e2_W2, e2_b2, s2_W, s2_b, n2_W, n2_b,
           cls_W, cls_b):
    src3d = edge_index[0].reshape(NSTEPS, KSUB, CH)
    dst3d = edge_index[1].reshape(NSTEPS, KSUB, CH)
    zeros_pad = jnp.zeros((N_ACC, D_IN), jnp.float32)

    ea_t = edge_attr.T
    attn1, attn2 = _attn_mlp2(ea_t,
                              e1_W1.T, e1_b1.reshape(-1, 1), e1_W2,
                              e1_b2.reshape(1, 1),
                              e2_W1.T, e2_b1.reshape(-1, 1), e2_W2,
                              e2_b2.reshape(1, 1))
    packed = src3d | (dst3d << DSHIFT)
    a1bits = lax.bitcast_convert_type(
        attn1, jnp.int32).reshape(NSTEPS, KSUB, CH)
    a2bits = lax.bitcast_convert_type(
        attn2, jnp.int32).reshape(NSTEPS, KSUB, CH)
    idx1 = jnp.concatenate([packed, a1bits], axis=1)
    idx2 = jnp.concatenate([packed, a2bits], axis=1)

    num1, den1 = _sc_gather_scatter(x, idx1, zeros_pad)
    h, hs2 = _layer_mid(num1, den1, x, s1_W, s1_b.reshape(1, -1),
                        n1_W, n1_b.reshape(1, -1),
                        s2_W, s2_b.reshape(1, -1))

    num2, den2 = _sc_gather_scatter(h, idx2, zeros_pad)
    logits = _layer_post(num2, den2, hs2, n2_W, n2_b.reshape(1, -1),
                         cls_W, cls_b.reshape(1, 1))
    return logits[:, 0]


# speedup vs baseline: 1.2918x; 1.0381x over previous
import functools

import jax
import jax.numpy as jnp
from jax import lax
from jax.experimental import pallas as pl
from jax.experimental.pallas import tpu as pltpu
from jax.experimental.pallas import tpu_sc as plsc

N_NODES = 10000
N_EDGES = 320000
D_IN = 128
NC, NS = 2, 16
NW = NC * NS
CH = 80
NCHUNKS = N_EDGES // CH
KSUB = 5
NSTEPS = NCHUNKS // KSUB
N_ACC = 10240
RPT = N_ACC // NS
N_DEN = 10000
LANES = 16
DSHIFT = 14


def _sc_gather_scatter(x, src3d, dst3d, attn3d, zeros_pad):
    mesh = plsc.VectorSubcoreMesh(core_axis_name="c", subcore_axis_name="s")

    @functools.partial(
        pl.kernel,
        out_type=(
            jax.ShapeDtypeStruct((NC, N_ACC, D_IN), jnp.float32),
            jax.ShapeDtypeStruct((NW, N_DEN), jnp.float32),
        ),
        mesh=mesh,
        scratch_types=[
            pltpu.VMEM((CH, D_IN), jnp.float32),
            pltpu.VMEM((CH, D_IN), jnp.float32),
            pltpu.VMEM((CH, D_IN), jnp.float32),
            pltpu.VMEM((N_DEN,), jnp.float32),
            pltpu.VMEM_SHARED((N_ACC, D_IN), jnp.float32),
            pltpu.SemaphoreType.DMA,
            pltpu.SemaphoreType.DMA,
            pltpu.SemaphoreType.DMA,
            pltpu.SemaphoreType.DMA,
            pltpu.SemaphoreType.DMA,
            pltpu.SemaphoreType.DMA,
        ],
        compiler_params=pltpu.CompilerParams(needs_layout_passes=False),
    )
    def k(x_hbm, src_hbm, dst_hbm, attn_hbm, zero_hbm, num_hbm, den_hbm,
          rows0, rows1, rows2, den_v, acc_sh,
          gs0, gs1, gs2, ss0, ss1, ss2):
        rows = (rows0, rows1, rows2)
        gsem = (gs0, gs1, gs2)
        ssem = (ss0, ss1, ss2)
        cid = lax.axis_index("c")
        sid = lax.axis_index("s")
        wid = cid * NS + sid

        pltpu.sync_copy(zero_hbm.at[pl.ds(sid * RPT, RPT)],
                        acc_sh.at[pl.ds(sid * RPT, RPT)])
        z16 = jnp.zeros((LANES,), jnp.float32)

        @pl.loop(0, N_DEN, step=LANES)
        def _(i):
            den_v[pl.ds(i, LANES)] = z16

        plsc.subcore_barrier()

        zi16 = jnp.zeros((LANES,), jnp.int32)

        def body(si_v, di_v, at_v):
            def scale(b, r):
                @pl.loop(0, CH, step=2)
                def _(c):
                    a0 = plsc.load_gather(
                        at_v, [zi16, jnp.full((LANES,), b, jnp.int32),
                               jnp.full((LANES,), c, jnp.int32)])
                    a1 = plsc.load_gather(
                        at_v, [zi16, jnp.full((LANES,), b, jnp.int32),
                               jnp.full((LANES,), c + 1, jnp.int32)])
                    for j in range(D_IN // LANES):
                        sl = pl.ds(j * LANES, LANES)
                        r[c, sl] = r[c, sl] * a0
                        r[c + 1, sl] = r[c + 1, sl] * a1

            def gath(b):
                return pltpu.async_copy(
                    x_hbm.at[si_v.at[0, b]], rows[b % 3], gsem[b % 3])

            def scat(b):
                return pltpu.async_copy(
                    rows[b % 3], acc_sh.at[di_v.at[0, b]], ssem[b % 3],
                    add=True)

            gd = [None] * KSUB
            sd = [None] * KSUB
            gd[0], gd[1], gd[2] = gath(0), gath(1), gath(2)

            for b in range(KSUB):
                for g in range(CH // LANES):
                    gsl = pl.ds(g * LANES, LANES)
                    plsc.addupdate_scatter(den_v, [di_v[0, b, gsl]],
                                           at_v[0, b, gsl])

            gd[0].wait(); scale(0, rows[0]); sd[0] = scat(0)
            for b in range(1, KSUB):
                gd[b].wait(); scale(b, rows[b % 3]); sd[b] = scat(b)
                nb = b + 2
                if 3 <= nb < KSUB:
                    sd[nb - 3].wait(); gd[nb] = gath(nb)
            for b in range(KSUB - 3, KSUB):
                sd[b].wait()

        pltpu.emit_pipeline(
            body,
            grid=(NSTEPS,),
            in_specs=[
                pl.BlockSpec((1, KSUB, CH), lambda i: (i, 0, 0)),
                pl.BlockSpec((1, KSUB, CH), lambda i: (i, 0, 0)),
                pl.BlockSpec((1, KSUB, CH), lambda i: (i, 0, 0)),
            ],
            out_specs=[],
            core_axis_name=("c", "s"),
            dimension_semantics=(pltpu.PARALLEL,),
        )(src_hbm, dst_hbm, attn_hbm)

        plsc.subcore_barrier()
        pltpu.sync_copy(acc_sh.at[pl.ds(sid * RPT, RPT)],
                        num_hbm.at[cid, pl.ds(sid * RPT, RPT)])
        pltpu.sync_copy(den_v, den_hbm.at[wid])

    return k(x, src3d, dst3d, attn3d, zeros_pad)


def _attn_mlp2(ea_t, W1t_a, b1c_a, W2c_a, b2_a, W1t_b, b1c_b, W2c_b, b2_b):
    BE = 32000

    def body(ea_ref, W1a, b1a, W2a, b2a, W1b, b1b, W2b, b2b, oa_ref, ob_ref):
        ea = ea_ref[...]
        for W1, b1, W2, b2, o_ref in ((W1a, b1a, W2a, b2a, oa_ref),
                                      (W1b, b1b, W2b, b2b, ob_ref)):
            h = jnp.maximum(
                jnp.dot(W1[...], ea,
                        preferred_element_type=jnp.float32) + b1[...], 0.0)
            z = jnp.sum(h * W2[...], axis=0, keepdims=True) + b2[...]
            o_ref[...] = 1.0 / (1.0 + jnp.exp(-z))

    wspecs = [
        pl.BlockSpec((32, 16), lambda i: (0, 0)),
        pl.BlockSpec((32, 1), lambda i: (0, 0)),
        pl.BlockSpec((32, 1), lambda i: (0, 0)),
        pl.BlockSpec((1, 1), lambda i: (0, 0)),
    ]
    return pl.pallas_call(
        body,
        grid=(N_EDGES // BE,),
        in_specs=[pl.BlockSpec((16, BE), lambda i: (0, i))] + wspecs + wspecs,
        out_specs=[pl.BlockSpec((1, BE), lambda i: (0, i))] * 2,
        out_shape=[jax.ShapeDtypeStruct((1, N_EDGES), jnp.float32)] * 2,
    )(ea_t, W1t_a, b1c_a, W2c_a, b2_a, W1t_b, b1c_b, W2c_b, b2_b)


BN = 1024


def _layer_mid(num_p, den_p, x, sW, sb, nW, nb, sW2, sb2):
    def body(a_ref, b_ref, dp_ref, x_ref, sW_ref, sb_ref, nW_ref, nb_ref,
             sW2_ref, sb2_ref, h_ref, hs_ref):
        den = jnp.sum(dp_ref[...], axis=0)[:, None] + 1e-8
        agg = (a_ref[0] + b_ref[0]) / den
        xs = jnp.dot(x_ref[...], sW_ref[...],
                     preferred_element_type=jnp.float32) + sb_ref[...]
        h = jnp.maximum(
            xs + jnp.dot(agg, nW_ref[...],
                         preferred_element_type=jnp.float32)
            + nb_ref[...], 0.0)
        h_ref[...] = h
        hs_ref[...] = jnp.dot(h, sW2_ref[...],
                              preferred_element_type=jnp.float32) + sb2_ref[...]

    return pl.pallas_call(
        body,
        grid=(pl.cdiv(N_NODES, BN),),
        in_specs=[
            pl.BlockSpec((1, BN, D_IN), lambda i: (0, i, 0)),
            pl.BlockSpec((1, BN, D_IN), lambda i: (1, i, 0)),
            pl.BlockSpec((NW, BN), lambda i: (0, i)),
            pl.BlockSpec((BN, D_IN), lambda i: (i, 0)),
            pl.BlockSpec((D_IN, D_IN), lambda i: (0, 0)),
            pl.BlockSpec((1, D_IN), lambda i: (0, 0)),
            pl.BlockSpec((D_IN, D_IN), lambda i: (0, 0)),
            pl.BlockSpec((1, D_IN), lambda i: (0, 0)),
            pl.BlockSpec((D_IN, D_IN), lambda i: (0, 0)),
            pl.BlockSpec((1, D_IN), lambda i: (0, 0)),
        ],
        out_specs=[
            pl.BlockSpec((BN, D_IN), lambda i: (i, 0)),
            pl.BlockSpec((BN, D_IN), lambda i: (i, 0)),
        ],
        out_shape=[
            jax.ShapeDtypeStruct((N_NODES, D_IN), jnp.float32),
            jax.ShapeDtypeStruct((N_NODES, D_IN), jnp.float32),
        ],
    )(num_p, num_p, den_p, x, sW, sb, nW, nb, sW2, sb2)


def _layer_post(num_p, den_p, hs, nW, nb, cW, cb):
    def body(a_ref, b_ref, dp_ref, hs_ref, nW_ref, nb_ref, cW_ref, cb_ref,
             o_ref):
        den = jnp.sum(dp_ref[...], axis=0)[:, None] + 1e-8
        agg = (a_ref[0] + b_ref[0]) / den
        h = jnp.maximum(
            hs_ref[...] + jnp.dot(agg, nW_ref[...],
                                  preferred_element_type=jnp.float32)
            + nb_ref[...], 0.0)
        o_ref[...] = jnp.dot(h, cW_ref[...],
                             preferred_element_type=jnp.float32) + cb_ref[...]

    return pl.pallas_call(
        body,
        grid=(pl.cdiv(N_NODES, BN),),
        in_specs=[
            pl.BlockSpec((1, BN, D_IN), lambda i: (0, i, 0)),
            pl.BlockSpec((1, BN, D_IN), lambda i: (1, i, 0)),
            pl.BlockSpec((NW, BN), lambda i: (0, i)),
            pl.BlockSpec((BN, D_IN), lambda i: (i, 0)),
            pl.BlockSpec((D_IN, D_IN), lambda i: (0, 0)),
            pl.BlockSpec((1, D_IN), lambda i: (0, 0)),
            pl.BlockSpec((D_IN, 1), lambda i: (0, 0)),
            pl.BlockSpec((1, 1), lambda i: (0, 0)),
        ],
        out_specs=pl.BlockSpec((BN, 1), lambda i: (i, 0)),
        out_shape=jax.ShapeDtypeStruct((N_NODES, 1), jnp.float32),
    )(num_p, num_p, den_p, hs, nW, nb, cW, cb)


def kernel(x, edge_index, edge_attr,
           e1_W1, e1_b1, e1_W2, e1_b2, s1_W, s1_b, n1_W, n1_b,
           e2_W1, e2_b1, e2_W2, e2_b2, s2_W, s2_b, n2_W, n2_b,
           cls_W, cls_b):
    src3d = edge_index[0].reshape(NSTEPS, KSUB, CH)
    dst3d = edge_index[1].reshape(NSTEPS, KSUB, CH)
    zeros_pad = jnp.zeros((N_ACC, D_IN), jnp.float32)

    ea_t = edge_attr.T
    attn1, attn2 = _attn_mlp2(ea_t,
                              e1_W1.T, e1_b1.reshape(-1, 1), e1_W2,
                              e1_b2.reshape(1, 1),
                              e2_W1.T, e2_b1.reshape(-1, 1), e2_W2,
                              e2_b2.reshape(1, 1))
    attn1_3d = attn1.reshape(NSTEPS, KSUB, CH)
    attn2_3d = attn2.reshape(NSTEPS, KSUB, CH)

    num1, den1 = _sc_gather_scatter(x, src3d, dst3d, attn1_3d, zeros_pad)
    h, hs2 = _layer_mid(num1, den1, x, s1_W, s1_b.reshape(1, -1),
                        n1_W, n1_b.reshape(1, -1),
                        s2_W, s2_b.reshape(1, -1))

    num2, den2 = _sc_gather_scatter(h, src3d, dst3d, attn2_3d, zeros_pad)
    logits = _layer_post(num2, den2, hs2, n2_W, n2_b.reshape(1, -1),
                         cls_W, cls_b.reshape(1, 1))
    return logits[:, 0]


# speedup vs baseline: 1.3239x; 1.0248x over previous
import functools

import jax
import jax.numpy as jnp
from jax import lax
from jax.experimental import pallas as pl
from jax.experimental.pallas import tpu as pltpu
from jax.experimental.pallas import tpu_sc as plsc

N_NODES = 10000
N_EDGES = 320000
D_IN = 128
NC, NS = 2, 16
NW = NC * NS
CH = 80
NCHUNKS = N_EDGES // CH
KSUB = 5
NSTEPS = NCHUNKS // KSUB
N_ACC = 10240
RPT = N_ACC // NS
N_DEN = 10000
LANES = 16
DSHIFT = 14


def _sc_gather_scatter(x, src3d, dst3d, attn3d, zeros_pad):
    mesh = plsc.VectorSubcoreMesh(core_axis_name="c", subcore_axis_name="s")

    @functools.partial(
        pl.kernel,
        out_type=(
            jax.ShapeDtypeStruct((NC, N_ACC, D_IN), jnp.float32),
            jax.ShapeDtypeStruct((NW, N_DEN), jnp.float32),
        ),
        mesh=mesh,
        scratch_types=[
            pltpu.VMEM((CH, D_IN), jnp.float32),
            pltpu.VMEM((CH, D_IN), jnp.float32),
            pltpu.VMEM((CH, D_IN), jnp.float32),
            pltpu.VMEM((N_DEN,), jnp.float32),
            pltpu.VMEM_SHARED((N_ACC, D_IN), jnp.float32),
            pltpu.SemaphoreType.DMA,
            pltpu.SemaphoreType.DMA,
            pltpu.SemaphoreType.DMA,
            pltpu.SemaphoreType.DMA,
            pltpu.SemaphoreType.DMA,
            pltpu.SemaphoreType.DMA,
        ],
        compiler_params=pltpu.CompilerParams(needs_layout_passes=False),
    )
    def k(x_hbm, src_hbm, dst_hbm, attn_hbm, zero_hbm, num_hbm, den_hbm,
          rows0, rows1, rows2, den_v, acc_sh,
          gs0, gs1, gs2, ss0, ss1, ss2):
        rows = (rows0, rows1, rows2)
        gsem = (gs0, gs1, gs2)
        ssem = (ss0, ss1, ss2)
        cid = lax.axis_index("c")
        sid = lax.axis_index("s")
        wid = cid * NS + sid

        pltpu.sync_copy(zero_hbm.at[pl.ds(sid * RPT, RPT)],
                        acc_sh.at[pl.ds(sid * RPT, RPT)])
        z16 = jnp.zeros((LANES,), jnp.float32)

        @pl.loop(0, N_DEN, step=LANES)
        def _(i):
            den_v[pl.ds(i, LANES)] = z16

        plsc.subcore_barrier()

        zi16 = jnp.zeros((LANES,), jnp.int32)

        def body(si_v, di_v, at_v):
            def scale(b, r):
                @pl.loop(0, CH, step=2)
                def _(c):
                    a0 = plsc.load_gather(
                        at_v, [zi16, jnp.full((LANES,), b, jnp.int32),
                               jnp.full((LANES,), c, jnp.int32)])
                    a1 = plsc.load_gather(
                        at_v, [zi16, jnp.full((LANES,), b, jnp.int32),
                               jnp.full((LANES,), c + 1, jnp.int32)])
                    for j in range(D_IN // LANES):
                        sl = pl.ds(j * LANES, LANES)
                        r[c, sl] = r[c, sl] * a0
                        r[c + 1, sl] = r[c + 1, sl] * a1

            def gath(b):
                return pltpu.async_copy(
                    x_hbm.at[si_v.at[0, b]], rows[b % 3], gsem[b % 3])

            def scat(b):
                return pltpu.async_copy(
                    rows[b % 3], acc_sh.at[di_v.at[0, b]], ssem[b % 3],
                    add=True)

            gd = [None] * KSUB
            sd = [None] * KSUB
            gd[0], gd[1], gd[2] = gath(0), gath(1), gath(2)

            for b in range(KSUB):
                for g in range(CH // LANES):
                    gsl = pl.ds(g * LANES, LANES)
                    plsc.addupdate_scatter(den_v, [di_v[0, b, gsl]],
                                           at_v[0, b, gsl])

            gd[0].wait(); scale(0, rows[0]); sd[0] = scat(0)
            for b in range(1, KSUB):
                gd[b].wait(); scale(b, rows[b % 3]); sd[b] = scat(b)
                nb = b + 2
                if 3 <= nb < KSUB:
                    sd[nb - 3].wait(); gd[nb] = gath(nb)
            for b in range(KSUB - 3, KSUB):
                sd[b].wait()

        pltpu.emit_pipeline(
            body,
            grid=(NSTEPS,),
            in_specs=[
                pl.BlockSpec((1, KSUB, CH), lambda i: (i, 0, 0)),
                pl.BlockSpec((1, KSUB, CH), lambda i: (i, 0, 0)),
                pl.BlockSpec((1, KSUB, CH), lambda i: (i, 0, 0)),
            ],
            out_specs=[],
            core_axis_name=("c", "s"),
            dimension_semantics=(pltpu.PARALLEL,),
        )(src_hbm, dst_hbm, attn_hbm)

        plsc.subcore_barrier()
        pltpu.sync_copy(acc_sh.at[pl.ds(sid * RPT, RPT)],
                        num_hbm.at[cid, pl.ds(sid * RPT, RPT)])
        pltpu.sync_copy(den_v, den_hbm.at[wid])

    return k(x, src3d, dst3d, attn3d, zeros_pad)


def _attn_mlp(ea_t, W1t, b1c, W2c, b2):
    BE = 32000

    def body(ea_ref, W1_ref, b1_ref, W2_ref, b2_ref, o_ref):
        h = jnp.maximum(
            jnp.dot(W1_ref[...], ea_ref[...],
                    preferred_element_type=jnp.float32) + b1_ref[...], 0.0)
        z = jnp.sum(h * W2_ref[...], axis=0, keepdims=True) + b2_ref[...]
        o_ref[...] = 1.0 / (1.0 + jnp.exp(-z))

    return pl.pallas_call(
        body,
        grid=(N_EDGES // BE,),
        in_specs=[
            pl.BlockSpec((16, BE), lambda i: (0, i)),
            pl.BlockSpec((32, 16), lambda i: (0, 0)),
            pl.BlockSpec((32, 1), lambda i: (0, 0)),
            pl.BlockSpec((32, 1), lambda i: (0, 0)),
            pl.BlockSpec((1, 1), lambda i: (0, 0)),
        ],
        out_specs=pl.BlockSpec((1, BE), lambda i: (0, i)),
        out_shape=jax.ShapeDtypeStruct((1, N_EDGES), jnp.float32),
    )(ea_t, W1t, b1c, W2c, b2)


BN = 1024


def _node_linear(x, sW, sb):
    def body(x_ref, sW_ref, sb_ref, xs_ref):
        xs_ref[...] = jnp.dot(x_ref[...], sW_ref[...],
                              preferred_element_type=jnp.float32) + sb_ref[...]

    return pl.pallas_call(
        body,
        grid=(pl.cdiv(N_NODES, BN),),
        in_specs=[
            pl.BlockSpec((BN, D_IN), lambda i: (i, 0)),
            pl.BlockSpec((D_IN, D_IN), lambda i: (0, 0)),
            pl.BlockSpec((1, D_IN), lambda i: (0, 0)),
        ],
        out_specs=pl.BlockSpec((BN, D_IN), lambda i: (i, 0)),
        out_shape=jax.ShapeDtypeStruct((N_NODES, D_IN), jnp.float32),
    )(x, sW, sb)


def _layer_mid(num_p, den_p, xs, nW, nb):
    def body(a_ref, b_ref, dp_ref, xs_ref, nW_ref, nb_ref, h_ref):
        den = jnp.sum(dp_ref[...], axis=0)[:, None] + 1e-8
        agg = (a_ref[0] + b_ref[0]) / den
        h_ref[...] = jnp.maximum(
            xs_ref[...] + jnp.dot(agg, nW_ref[...],
                                  preferred_element_type=jnp.float32)
            + nb_ref[...], 0.0)

    return pl.pallas_call(
        body,
        grid=(pl.cdiv(N_NODES, BN),),
        in_specs=[
            pl.BlockSpec((1, BN, D_IN), lambda i: (0, i, 0)),
            pl.BlockSpec((1, BN, D_IN), lambda i: (1, i, 0)),
            pl.BlockSpec((NW, BN), lambda i: (0, i)),
            pl.BlockSpec((BN, D_IN), lambda i: (i, 0)),
            pl.BlockSpec((D_IN, D_IN), lambda i: (0, 0)),
            pl.BlockSpec((1, D_IN), lambda i: (0, 0)),
        ],
        out_specs=pl.BlockSpec((BN, D_IN), lambda i: (i, 0)),
        out_shape=jax.ShapeDtypeStruct((N_NODES, D_IN), jnp.float32),
    )(num_p, num_p, den_p, xs, nW, nb)


def _layer_post(num_p, den_p, hs, nW, nb, cW, cb):
    def body(a_ref, b_ref, dp_ref, hs_ref, nW_ref, nb_ref, cW_ref, cb_ref,
             o_ref):
        den = jnp.sum(dp_ref[...], axis=0)[:, None] + 1e-8
        agg = (a_ref[0] + b_ref[0]) / den
        h = jnp.maximum(
            hs_ref[...] + jnp.dot(agg, nW_ref[...],
                                  preferred_element_type=jnp.float32)
            + nb_ref[...], 0.0)
        o_ref[...] = jnp.dot(h, cW_ref[...],
                             preferred_element_type=jnp.float32) + cb_ref[...]

    return pl.pallas_call(
        body,
        grid=(pl.cdiv(N_NODES, BN),),
        in_specs=[
            pl.BlockSpec((1, BN, D_IN), lambda i: (0, i, 0)),
            pl.BlockSpec((1, BN, D_IN), lambda i: (1, i, 0)),
            pl.BlockSpec((NW, BN), lambda i: (0, i)),
            pl.BlockSpec((BN, D_IN), lambda i: (i, 0)),
            pl.BlockSpec((D_IN, D_IN), lambda i: (0, 0)),
            pl.BlockSpec((1, D_IN), lambda i: (0, 0)),
            pl.BlockSpec((D_IN, 1), lambda i: (0, 0)),
            pl.BlockSpec((1, 1), lambda i: (0, 0)),
        ],
        out_specs=pl.BlockSpec((BN, 1), lambda i: (i, 0)),
        out_shape=jax.ShapeDtypeStruct((N_NODES, 1), jnp.float32),
    )(num_p, num_p, den_p, hs, nW, nb, cW, cb)


def kernel(x, edge_index, edge_attr,
           e1_W1, e1_b1, e1_W2, e1_b2, s1_W, s1_b, n1_W, n1_b,
           e2_W1, e2_b1, e2_W2, e2_b2, s2_W, s2_b, n2_W, n2_b,
           cls_W, cls_b):
    src3d = edge_index[0].reshape(NSTEPS, KSUB, CH)
    dst3d = edge_index[1].reshape(NSTEPS, KSUB, CH)
    zeros_pad = jnp.zeros((N_ACC, D_IN), jnp.float32)

    ea_t = edge_attr.T
    attn1 = _attn_mlp(ea_t, e1_W1.T, e1_b1.reshape(-1, 1),
                      e1_W2, e1_b2.reshape(1, 1)).reshape(NSTEPS, KSUB, CH)
    attn2 = _attn_mlp(ea_t, e2_W1.T, e2_b1.reshape(-1, 1),
                      e2_W2, e2_b2.reshape(1, 1)).reshape(NSTEPS, KSUB, CH)
    xs1 = _node_linear(x, s1_W, s1_b.reshape(1, -1))

    num1, den1 = _sc_gather_scatter(x, src3d, dst3d, attn1, zeros_pad)
    h = _layer_mid(num1, den1, xs1, n1_W, n1_b.reshape(1, -1))
    hs2 = _node_linear(h, s2_W, s2_b.reshape(1, -1))

    num2, den2 = _sc_gather_scatter(h, src3d, dst3d, attn2, zeros_pad)
    logits = _layer_post(num2, den2, hs2, n2_W, n2_b.reshape(1, -1),
                         cls_W, cls_b.reshape(1, 1))
    return logits[:, 0]


# speedup vs baseline: 1.3670x; 1.0326x over previous
import functools

import jax
import jax.numpy as jnp
from jax import lax
from jax.experimental import pallas as pl
from jax.experimental.pallas import tpu as pltpu
from jax.experimental.pallas import tpu_sc as plsc

N_NODES = 10000
N_EDGES = 320000
D_IN = 128
NC, NS = 2, 16
NW = NC * NS
CH = 80
NCHUNKS = N_EDGES // CH
KSUB = 5
NSTEPS = NCHUNKS // KSUB
N_ACC = 10240
RPT = N_ACC // NS
N_DEN = 10000
LANES = 16
DSHIFT = 14


def _sc_gather_scatter(x, src3d, dst3d, attn3d, zeros_pad):
    mesh = plsc.VectorSubcoreMesh(core_axis_name="c", subcore_axis_name="s")

    @functools.partial(
        pl.kernel,
        out_type=(
            jax.ShapeDtypeStruct((NC, N_ACC, D_IN), jnp.float32),
            jax.ShapeDtypeStruct((NW, N_DEN), jnp.float32),
        ),
        mesh=mesh,
        scratch_types=[
            pltpu.VMEM((CH, D_IN), jnp.float32),
            pltpu.VMEM((CH, D_IN), jnp.float32),
            pltpu.VMEM((CH, D_IN), jnp.float32),
            pltpu.VMEM((N_DEN,), jnp.float32),
            pltpu.VMEM_SHARED((N_ACC, D_IN), jnp.float32),
            pltpu.SemaphoreType.DMA,
            pltpu.SemaphoreType.DMA,
            pltpu.SemaphoreType.DMA,
            pltpu.SemaphoreType.DMA,
            pltpu.SemaphoreType.DMA,
            pltpu.SemaphoreType.DMA,
        ],
        compiler_params=pltpu.CompilerParams(needs_layout_passes=False),
    )
    def k(x_hbm, src_hbm, dst_hbm, attn_hbm, zero_hbm, num_hbm, den_hbm,
          rows0, rows1, rows2, den_v, acc_sh,
          gs0, gs1, gs2, ss0, ss1, ss2):
        rows = (rows0, rows1, rows2)
        gsem = (gs0, gs1, gs2)
        ssem = (ss0, ss1, ss2)
        cid = lax.axis_index("c")
        sid = lax.axis_index("s")
        wid = cid * NS + sid

        pltpu.sync_copy(zero_hbm.at[pl.ds(sid * RPT, RPT)],
                        acc_sh.at[pl.ds(sid * RPT, RPT)])
        z16 = jnp.zeros((LANES,), jnp.float32)

        @pl.loop(0, N_DEN, step=LANES)
        def _(i):
            den_v[pl.ds(i, LANES)] = z16

        plsc.subcore_barrier()

        zi16 = jnp.zeros((LANES,), jnp.int32)

        def body(si_v, di_v, at_v):
            def scale(b, r):
                @pl.loop(0, CH, step=4)
                def _(c):
                    bvec = jnp.full((LANES,), b, jnp.int32)
                    avs = [plsc.load_gather(
                        at_v, [zi16, bvec,
                               jnp.full((LANES,), c + u, jnp.int32)])
                        for u in range(4)]
                    for j in range(D_IN // LANES):
                        sl = pl.ds(j * LANES, LANES)
                        for u in range(4):
                            r[c + u, sl] = r[c + u, sl] * avs[u]

            def gath(b):
                return pltpu.async_copy(
                    x_hbm.at[si_v.at[0, b]], rows[b % 3], gsem[b % 3])

            def scat(b):
                return pltpu.async_copy(
                    rows[b % 3], acc_sh.at[di_v.at[0, b]], ssem[b % 3],
                    add=True)

            gd = [None] * KSUB
            sd = [None] * KSUB
            gd[0], gd[1], gd[2] = gath(0), gath(1), gath(2)

            for b in range(KSUB):
                for g in range(CH // LANES):
                    gsl = pl.ds(g * LANES, LANES)
                    plsc.addupdate_scatter(den_v, [di_v[0, b, gsl]],
                                           at_v[0, b, gsl])

            gd[0].wait(); scale(0, rows[0]); sd[0] = scat(0)
            for b in range(1, KSUB):
                gd[b].wait(); scale(b, rows[b % 3]); sd[b] = scat(b)
                nb = b + 2
                if 3 <= nb < KSUB:
                    sd[nb - 3].wait(); gd[nb] = gath(nb)
            for b in range(KSUB - 3, KSUB):
                sd[b].wait()

        pltpu.emit_pipeline(
            body,
            grid=(NSTEPS,),
            in_specs=[
                pl.BlockSpec((1, KSUB, CH), lambda i: (i, 0, 0)),
                pl.BlockSpec((1, KSUB, CH), lambda i: (i, 0, 0)),
                pl.BlockSpec((1, KSUB, CH), lambda i: (i, 0, 0)),
            ],
            out_specs=[],
            core_axis_name=("c", "s"),
            dimension_semantics=(pltpu.PARALLEL,),
        )(src_hbm, dst_hbm, attn_hbm)

        plsc.subcore_barrier()
        pltpu.sync_copy(acc_sh.at[pl.ds(sid * RPT, RPT)],
                        num_hbm.at[cid, pl.ds(sid * RPT, RPT)])
        pltpu.sync_copy(den_v, den_hbm.at[wid])

    return k(x, src3d, dst3d, attn3d, zeros_pad)


def _attn_mlp(ea_t, W1t, b1c, W2c, b2):
    BE = 32000

    def body(ea_ref, W1_ref, b1_ref, W2_ref, b2_ref, o_ref):
        h = jnp.maximum(
            jnp.dot(W1_ref[...], ea_ref[...],
                    preferred_element_type=jnp.float32) + b1_ref[...], 0.0)
        z = jnp.sum(h * W2_ref[...], axis=0, keepdims=True) + b2_ref[...]
        o_ref[...] = 1.0 / (1.0 + jnp.exp(-z))

    return pl.pallas_call(
        body,
        grid=(N_EDGES // BE,),
        in_specs=[
            pl.BlockSpec((16, BE), lambda i: (0, i)),
            pl.BlockSpec((32, 16), lambda i: (0, 0)),
            pl.BlockSpec((32, 1), lambda i: (0, 0)),
            pl.BlockSpec((32, 1), lambda i: (0, 0)),
            pl.BlockSpec((1, 1), lambda i: (0, 0)),
        ],
        out_specs=pl.BlockSpec((1, BE), lambda i: (0, i)),
        out_shape=jax.ShapeDtypeStruct((1, N_EDGES), jnp.float32),
    )(ea_t, W1t, b1c, W2c, b2)


BN = 1024


def _node_linear(x, sW, sb):
    def body(x_ref, sW_ref, sb_ref, xs_ref):
        xs_ref[...] = jnp.dot(x_ref[...], sW_ref[...],
                              preferred_element_type=jnp.float32) + sb_ref[...]

    return pl.pallas_call(
        body,
        grid=(pl.cdiv(N_NODES, BN),),
        in_specs=[
            pl.BlockSpec((BN, D_IN), lambda i: (i, 0)),
            pl.BlockSpec((D_IN, D_IN), lambda i: (0, 0)),
            pl.BlockSpec((1, D_IN), lambda i: (0, 0)),
        ],
        out_specs=pl.BlockSpec((BN, D_IN), lambda i: (i, 0)),
        out_shape=jax.ShapeDtypeStruct((N_NODES, D_IN), jnp.float32),
    )(x, sW, sb)


def _layer_mid(num_p, den_p, xs, nW, nb):
    def body(a_ref, b_ref, dp_ref, xs_ref, nW_ref, nb_ref, h_ref):
        den = jnp.sum(dp_ref[...], axis=0)[:, None] + 1e-8
        agg = (a_ref[0] + b_ref[0]) / den
        h_ref[...] = jnp.maximum(
            xs_ref[...] + jnp.dot(agg, nW_ref[...],
                                  preferred_element_type=jnp.float32)
            + nb_ref[...], 0.0)

    return pl.pallas_call(
        body,
        grid=(pl.cdiv(N_NODES, BN),),
        in_specs=[
            pl.BlockSpec((1, BN, D_IN), lambda i: (0, i, 0)),
            pl.BlockSpec((1, BN, D_IN), lambda i: (1, i, 0)),
            pl.BlockSpec((NW, BN), lambda i: (0, i)),
            pl.BlockSpec((BN, D_IN), lambda i: (i, 0)),
            pl.BlockSpec((D_IN, D_IN), lambda i: (0, 0)),
            pl.BlockSpec((1, D_IN), lambda i: (0, 0)),
        ],
        out_specs=pl.BlockSpec((BN, D_IN), lambda i: (i, 0)),
        out_shape=jax.ShapeDtypeStruct((N_NODES, D_IN), jnp.float32),
    )(num_p, num_p, den_p, xs, nW, nb)


def _layer_post(num_p, den_p, hs, nW, nb, cW, cb):
    def body(a_ref, b_ref, dp_ref, hs_ref, nW_ref, nb_ref, cW_ref, cb_ref,
             o_ref):
        den = jnp.sum(dp_ref[...], axis=0)[:, None] + 1e-8
        agg = (a_ref[0] + b_ref[0]) / den
        h = jnp.maximum(
            hs_ref[...] + jnp.dot(agg, nW_ref[...],
                                  preferred_element_type=jnp.float32)
            + nb_ref[...], 0.0)
        o_ref[...] = jnp.dot(h, cW_ref[...],
                             preferred_element_type=jnp.float32) + cb_ref[...]

    return pl.pallas_call(
        body,
        grid=(pl.cdiv(N_NODES, BN),),
        in_specs=[
            pl.BlockSpec((1, BN, D_IN), lambda i: (0, i, 0)),
            pl.BlockSpec((1, BN, D_IN), lambda i: (1, i, 0)),
            pl.BlockSpec((NW, BN), lambda i: (0, i)),
            pl.BlockSpec((BN, D_IN), lambda i: (i, 0)),
            pl.BlockSpec((D_IN, D_IN), lambda i: (0, 0)),
            pl.BlockSpec((1, D_IN), lambda i: (0, 0)),
            pl.BlockSpec((D_IN, 1), lambda i: (0, 0)),
            pl.BlockSpec((1, 1), lambda i: (0, 0)),
        ],
        out_specs=pl.BlockSpec((BN, 1), lambda i: (i, 0)),
        out_shape=jax.ShapeDtypeStruct((N_NODES, 1), jnp.float32),
    )(num_p, num_p, den_p, hs, nW, nb, cW, cb)


def kernel(x, edge_index, edge_attr,
           e1_W1, e1_b1, e1_W2, e1_b2, s1_W, s1_b, n1_W, n1_b,
           e2_W1, e2_b1, e2_W2, e2_b2, s2_W, s2_b, n2_W, n2_b,
           cls_W, cls_b):
    src3d = edge_index[0].reshape(NSTEPS, KSUB, CH)
    dst3d = edge_index[1].reshape(NSTEPS, KSUB, CH)
    zeros_pad = jnp.zeros((N_ACC, D_IN), jnp.float32)

    ea_t = edge_attr.T
    attn1 = _attn_mlp(ea_t, e1_W1.T, e1_b1.reshape(-1, 1),
                      e1_W2, e1_b2.reshape(1, 1)).reshape(NSTEPS, KSUB, CH)
    attn2 = _attn_mlp(ea_t, e2_W1.T, e2_b1.reshape(-1, 1),
                      e2_W2, e2_b2.reshape(1, 1)).reshape(NSTEPS, KSUB, CH)
    xs1 = _node_linear(x, s1_W, s1_b.reshape(1, -1))

    num1, den1 = _sc_gather_scatter(x, src3d, dst3d, attn1, zeros_pad)
    h = _layer_mid(num1, den1, xs1, n1_W, n1_b.reshape(1, -1))
    hs2 = _node_linear(h, s2_W, s2_b.reshape(1, -1))

    num2, den2 = _sc_gather_scatter(h, src3d, dst3d, attn2, zeros_pad)
    logits = _layer_post(num2, den2, hs2, n2_W, n2_b.reshape(1, -1),
                         cls_W, cls_b.reshape(1, 1))
    return logits[:, 0]


# speedup vs baseline: 1.3703x; 1.0024x over previous
import functools

import jax
import jax.numpy as jnp
from jax import lax
from jax.experimental import pallas as pl
from jax.experimental.pallas import tpu as pltpu
from jax.experimental.pallas import tpu_sc as plsc

N_NODES = 10000
N_EDGES = 320000
D_IN = 128
NC, NS = 2, 16
NW = NC * NS
CH = 80
NCHUNKS = N_EDGES // CH
KSUB = 5
NSTEPS = NCHUNKS // KSUB
N_ACC = 10240
RPT = N_ACC // NS
N_DEN = 10000
LANES = 16
DSHIFT = 14


def _sc_gather_scatter(x, src3d, dst3d, attn3d, zeros_pad):
    mesh = plsc.VectorSubcoreMesh(core_axis_name="c", subcore_axis_name="s")

    @functools.partial(
        pl.kernel,
        out_type=(
            jax.ShapeDtypeStruct((NC, N_ACC, D_IN), jnp.float32),
            jax.ShapeDtypeStruct((NW, N_DEN), jnp.float32),
        ),
        mesh=mesh,
        scratch_types=[
            pltpu.VMEM((CH, D_IN), jnp.float32),
            pltpu.VMEM((CH, D_IN), jnp.float32),
            pltpu.VMEM((CH, D_IN), jnp.float32),
            pltpu.VMEM((N_DEN,), jnp.float32),
            pltpu.VMEM_SHARED((N_ACC, D_IN), jnp.float32),
            pltpu.SemaphoreType.DMA,
            pltpu.SemaphoreType.DMA,
            pltpu.SemaphoreType.DMA,
            pltpu.SemaphoreType.DMA,
            pltpu.SemaphoreType.DMA,
            pltpu.SemaphoreType.DMA,
        ],
        compiler_params=pltpu.CompilerParams(needs_layout_passes=False),
    )
    def k(x_hbm, src_hbm, dst_hbm, attn_hbm, zero_hbm, num_hbm, den_hbm,
          rows0, rows1, rows2, den_v, acc_sh,
          gs0, gs1, gs2, ss0, ss1, ss2):
        rows = (rows0, rows1, rows2)
        gsem = (gs0, gs1, gs2)
        ssem = (ss0, ss1, ss2)
        cid = lax.axis_index("c")
        sid = lax.axis_index("s")
        wid = cid * NS + sid

        pltpu.sync_copy(zero_hbm.at[pl.ds(sid * RPT, RPT)],
                        acc_sh.at[pl.ds(sid * RPT, RPT)])
        z16 = jnp.zeros((LANES,), jnp.float32)

        @pl.loop(0, N_DEN, step=LANES)
        def _(i):
            den_v[pl.ds(i, LANES)] = z16

        plsc.subcore_barrier()

        zi16 = jnp.zeros((LANES,), jnp.int32)

        def body(si_v, di_v, at_v):
            def scale(b, r):
                @pl.loop(0, CH, step=8)
                def _(c):
                    bvec = jnp.full((LANES,), b, jnp.int32)
                    avs = [plsc.load_gather(
                        at_v, [zi16, bvec,
                               jnp.full((LANES,), c + u, jnp.int32)])
                        for u in range(8)]
                    for j in range(D_IN // LANES):
                        sl = pl.ds(j * LANES, LANES)
                        for u in range(8):
                            r[c + u, sl] = r[c + u, sl] * avs[u]

            def gath(b):
                return pltpu.async_copy(
                    x_hbm.at[si_v.at[0, b]], rows[b % 3], gsem[b % 3])

            def scat(b):
                return pltpu.async_copy(
                    rows[b % 3], acc_sh.at[di_v.at[0, b]], ssem[b % 3],
                    add=True)

            gd = [None] * KSUB
            sd = [None] * KSUB
            gd[0], gd[1], gd[2] = gath(0), gath(1), gath(2)

            for b in range(KSUB):
                for g in range(CH // LANES):
                    gsl = pl.ds(g * LANES, LANES)
                    plsc.addupdate_scatter(den_v, [di_v[0, b, gsl]],
                                           at_v[0, b, gsl])

            gd[0].wait(); scale(0, rows[0]); sd[0] = scat(0)
            for b in range(1, KSUB):
                gd[b].wait(); scale(b, rows[b % 3]); sd[b] = scat(b)
                nb = b + 2
                if 3 <= nb < KSUB:
                    sd[nb - 3].wait(); gd[nb] = gath(nb)
            for b in range(KSUB - 3, KSUB):
                sd[b].wait()

        pltpu.emit_pipeline(
            body,
            grid=(NSTEPS,),
            in_specs=[
                pl.BlockSpec((1, KSUB, CH), lambda i: (i, 0, 0)),
                pl.BlockSpec((1, KSUB, CH), lambda i: (i, 0, 0)),
                pl.BlockSpec((1, KSUB, CH), lambda i: (i, 0, 0)),
            ],
            out_specs=[],
            core_axis_name=("c", "s"),
            dimension_semantics=(pltpu.PARALLEL,),
        )(src_hbm, dst_hbm, attn_hbm)

        plsc.subcore_barrier()
        pltpu.sync_copy(acc_sh.at[pl.ds(sid * RPT, RPT)],
                        num_hbm.at[cid, pl.ds(sid * RPT, RPT)])
        pltpu.sync_copy(den_v, den_hbm.at[wid])

    return k(x, src3d, dst3d, attn3d, zeros_pad)


def _attn_mlp(ea_t, W1t, b1c, W2c, b2):
    BE = 32000

    def body(ea_ref, W1_ref, b1_ref, W2_ref, b2_ref, o_ref):
        h = jnp.maximum(
            jnp.dot(W1_ref[...], ea_ref[...],
                    preferred_element_type=jnp.float32) + b1_ref[...], 0.0)
        z = jnp.sum(h * W2_ref[...], axis=0, keepdims=True) + b2_ref[...]
        o_ref[...] = 1.0 / (1.0 + jnp.exp(-z))

    return pl.pallas_call(
        body,
        grid=(N_EDGES // BE,),
        in_specs=[
            pl.BlockSpec((16, BE), lambda i: (0, i)),
            pl.BlockSpec((32, 16), lambda i: (0, 0)),
            pl.BlockSpec((32, 1), lambda i: (0, 0)),
            pl.BlockSpec((32, 1), lambda i: (0, 0)),
            pl.BlockSpec((1, 1), lambda i: (0, 0)),
        ],
        out_specs=pl.BlockSpec((1, BE), lambda i: (0, i)),
        out_shape=jax.ShapeDtypeStruct((1, N_EDGES), jnp.float32),
    )(ea_t, W1t, b1c, W2c, b2)


BN = 1024


def _node_linear(x, sW, sb):
    def body(x_ref, sW_ref, sb_ref, xs_ref):
        xs_ref[...] = jnp.dot(x_ref[...], sW_ref[...],
                              preferred_element_type=jnp.float32) + sb_ref[...]

    return pl.pallas_call(
        body,
        grid=(pl.cdiv(N_NODES, BN),),
        in_specs=[
            pl.BlockSpec((BN, D_IN), lambda i: (i, 0)),
            pl.BlockSpec((D_IN, D_IN), lambda i: (0, 0)),
            pl.BlockSpec((1, D_IN), lambda i: (0, 0)),
        ],
        out_specs=pl.BlockSpec((BN, D_IN), lambda i: (i, 0)),
        out_shape=jax.ShapeDtypeStruct((N_NODES, D_IN), jnp.float32),
    )(x, sW, sb)


def _layer_mid(num_p, den_p, xs, nW, nb):
    def body(a_ref, b_ref, dp_ref, xs_ref, nW_ref, nb_ref, h_ref):
        den = jnp.sum(dp_ref[...], axis=0)[:, None] + 1e-8
        agg = (a_ref[0] + b_ref[0]) / den
        h_ref[...] = jnp.maximum(
            xs_ref[...] + jnp.dot(agg, nW_ref[...],
                                  preferred_element_type=jnp.float32)
            + nb_ref[...], 0.0)

    return pl.pallas_call(
        body,
        grid=(pl.cdiv(N_NODES, BN),),
        in_specs=[
            pl.BlockSpec((1, BN, D_IN), lambda i: (0, i, 0)),
            pl.BlockSpec((1, BN, D_IN), lambda i: (1, i, 0)),
            pl.BlockSpec((NW, BN), lambda i: (0, i)),
            pl.BlockSpec((BN, D_IN), lambda i: (i, 0)),
            pl.BlockSpec((D_IN, D_IN), lambda i: (0, 0)),
            pl.BlockSpec((1, D_IN), lambda i: (0, 0)),
        ],
        out_specs=pl.BlockSpec((BN, D_IN), lambda i: (i, 0)),
        out_shape=jax.ShapeDtypeStruct((N_NODES, D_IN), jnp.float32),
    )(num_p, num_p, den_p, xs, nW, nb)


def _layer_post(num_p, den_p, hs, nW, nb, cW, cb):
    def body(a_ref, b_ref, dp_ref, hs_ref, nW_ref, nb_ref, cW_ref, cb_ref,
             o_ref):
        den = jnp.sum(dp_ref[...], axis=0)[:, None] + 1e-8
        agg = (a_ref[0] + b_ref[0]) / den
        h = jnp.maximum(
            hs_ref[...] + jnp.dot(agg, nW_ref[...],
                                  preferred_element_type=jnp.float32)
            + nb_ref[...], 0.0)
        o_ref[...] = jnp.dot(h, cW_ref[...],
                             preferred_element_type=jnp.float32) + cb_ref[...]

    return pl.pallas_call(
        body,
        grid=(pl.cdiv(N_NODES, BN),),
        in_specs=[
            pl.BlockSpec((1, BN, D_IN), lambda i: (0, i, 0)),
            pl.BlockSpec((1, BN, D_IN), lambda i: (1, i, 0)),
            pl.BlockSpec((NW, BN), lambda i: (0, i)),
            pl.BlockSpec((BN, D_IN), lambda i: (i, 0)),
            pl.BlockSpec((D_IN, D_IN), lambda i: (0, 0)),
            pl.BlockSpec((1, D_IN), lambda i: (0, 0)),
            pl.BlockSpec((D_IN, 1), lambda i: (0, 0)),
            pl.BlockSpec((1, 1), lambda i: (0, 0)),
        ],
        out_specs=pl.BlockSpec((BN, 1), lambda i: (i, 0)),
        out_shape=jax.ShapeDtypeStruct((N_NODES, 1), jnp.float32),
    )(num_p, num_p, den_p, hs, nW, nb, cW, cb)


def kernel(x, edge_index, edge_attr,
           e1_W1, e1_b1, e1_W2, e1_b2, s1_W, s1_b, n1_W, n1_b,
           e2_W1, e2_b1, e2_W2, e2_b2, s2_W, s2_b, n2_W, n2_b,
           cls_W, cls_b):
    src3d = edge_index[0].reshape(NSTEPS, KSUB, CH)
    dst3d = edge_index[1].reshape(NSTEPS, KSUB, CH)
    zeros_pad = jnp.zeros((N_ACC, D_IN), jnp.float32)

    ea_t = edge_attr.T
    attn1 = _attn_mlp(ea_t, e1_W1.T, e1_b1.reshape(-1, 1),
                      e1_W2, e1_b2.reshape(1, 1)).reshape(NSTEPS, KSUB, CH)
    attn2 = _attn_mlp(ea_t, e2_W1.T, e2_b1.reshape(-1, 1),
                      e2_W2, e2_b2.reshape(1, 1)).reshape(NSTEPS, KSUB, CH)
    xs1 = _node_linear(x, s1_W, s1_b.reshape(1, -1))

    num1, den1 = _sc_gather_scatter(x, src3d, dst3d, attn1, zeros_pad)
    h = _layer_mid(num1, den1, xs1, n1_W, n1_b.reshape(1, -1))
    hs2 = _node_linear(h, s2_W, s2_b.reshape(1, -1))

    num2, den2 = _sc_gather_scatter(h, src3d, dst3d, attn2, zeros_pad)
    logits = _layer_post(num2, den2, hs2, n2_W, n2_b.reshape(1, -1),
                         cls_W, cls_b.reshape(1, 1))
    return logits[:, 0]
